# baseline (speedup 1.0000x reference)
import functools

import jax
import jax.numpy as jnp
from jax import lax
from jax.experimental import pallas as pl
from jax.experimental.pallas import tpu as pltpu
from jax.experimental.pallas import tpu_sc as plsc

_B, _L, _K = 64, 2048, 256
_THR = 0.5
_BIG = 1 << 30


_PRE = 8


def _wta_body(x_ref, idx_ref, w_ref, y_ref, idx_s):
    x = x_ref[0]
    kk1 = lax.broadcasted_iota(jnp.int32, (1, _K), 1)
    xp = x[0:_PRE, :]
    iip = lax.broadcasted_iota(jnp.int32, (_PRE, _K), 0)
    kkp = lax.broadcasted_iota(jnp.int32, (_PRE, _K), 1)
    ffp = jnp.min(jnp.where(xp > _THR, iip * _K + kkp, _BIG))

    @pl.when(ffp < _BIG)
    def _():
        idx_s[0] = lax.rem(ffp, _K)

    @pl.when(ffp >= _BIG)
    def _():
        s = x > _THR
        ii = lax.broadcasted_iota(jnp.int32, (_L, _K), 0)
        kk = lax.broadcasted_iota(jnp.int32, (_L, _K), 1)
        ff = jnp.min(jnp.where(s, ii * _K + kk, _BIG))
        total = jnp.sum(x, axis=0, keepdims=True)
        maxv = jnp.max(total)
        k_fb = jnp.min(jnp.where(total == maxv, kk1, _BIG))
        idx_s[0] = jnp.where(ff < _BIG, lax.rem(ff, _K), k_fb)

    idx = idx_s[0]
    w = (kk1 == idx).astype(jnp.float32)
    idx_ref[0] = jnp.full((1, 1), idx, jnp.int32)
    w_ref[0] = w
    y_ref[0] = x * w


def _full_path(spikes):
    idx3, w3, y = pl.pallas_call(
        _wta_body,
        grid=(_B,),
        in_specs=[pl.BlockSpec((1, _L, _K), lambda b: (b, 0, 0))],
        out_specs=[
            pl.BlockSpec((1, 1, 1), lambda b: (b, 0, 0)),
            pl.BlockSpec((1, 1, _K), lambda b: (b, 0, 0)),
            pl.BlockSpec((1, _L, _K), lambda b: (b, 0, 0)),
        ],
        out_shape=[
            jax.ShapeDtypeStruct((_B, 1, 1), jnp.int32),
            jax.ShapeDtypeStruct((_B, 1, _K), jnp.float32),
            jax.ShapeDtypeStruct((_B, _L, _K), jnp.float32),
        ],
        scratch_shapes=[pltpu.SMEM((1,), jnp.int32)],
    )(spikes)
    return idx3[:, 0, 0], w3[:, 0, :], y


def _prefix_body(x_ref, ff_ref, idx_ref, w_ref):
    x = x_ref[...]
    ii = lax.broadcasted_iota(jnp.int32, (_B, _PRE, _K), 1)
    kk = lax.broadcasted_iota(jnp.int32, (_B, _PRE, _K), 2)
    ff = jnp.min(jnp.where(x > _THR, ii * _K + kk, _BIG), axis=(1, 2))
    ff_ref[...] = ff.reshape(_B, 1, 1)
    idx = lax.rem(ff, _K).reshape(_B, 1, 1)
    idx_ref[...] = idx
    kk3 = lax.broadcasted_iota(jnp.int32, (_B, 1, _K), 2)
    w_ref[...] = (kk3 == idx).astype(jnp.float32)


_GRPS = _K // 128


def _mask_body(idxp_ref, x_ref, idx_ref, w_ref, y_ref):
    b = pl.program_id(0)
    idx = idxp_ref[b]
    grp = idx // 128
    base = pl.multiple_of(grp * 128, 128)
    obase = pl.multiple_of((1 - grp) * 128, 128)
    lanei = lax.broadcasted_iota(jnp.int32, (1, 128), 1)
    wrow = (lanei == idx - base).astype(jnp.float32)
    y_ref[0, :, pl.ds(obase, 128)] = jnp.zeros((_L, 128), jnp.float32)
    y_ref[0, :, pl.ds(base, 128)] = x_ref[0] * wrow
    w_ref[0, :, pl.ds(obase, 128)] = jnp.zeros((1, 128), jnp.float32)
    w_ref[0, :, pl.ds(base, 128)] = wrow
    idx_ref[0] = jnp.full((1, 1), idx, jnp.int32)


def _cheap_path(spikes, idxv):
    grid_spec = pltpu.PrefetchScalarGridSpec(
        num_scalar_prefetch=1,
        grid=(_B,),
        in_specs=[
            pl.BlockSpec((1, _L, 128), lambda b, idxp: (b, 0, idxp[b] // 128)),
        ],
        out_specs=[
            pl.BlockSpec((1, 1, 1), lambda b, idxp: (b, 0, 0)),
            pl.BlockSpec((1, 1, _K), lambda b, idxp: (b, 0, 0)),
            pl.BlockSpec((1, _L, _K), lambda b, idxp: (b, 0, 0)),
        ],
    )
    idx3, w3, y = pl.pallas_call(
        _mask_body,
        grid_spec=grid_spec,
        out_shape=[
            jax.ShapeDtypeStruct((_B, 1, 1), jnp.int32),
            jax.ShapeDtypeStruct((_B, 1, _K), jnp.float32),
            jax.ShapeDtypeStruct((_B, _L, _K), jnp.float32),
        ],
    )(idxv, spikes)
    return idx3[:, 0, 0], w3[:, 0, :], y


_NW = 32
_TROWS = 128
_NCHUNK = _L // _TROWS


def _sc_y_body(x_hbm, idx_hbm, y_hbm, ibuf, xbuf, vslab, zslab):
    wid = lax.axis_index("s") * 2 + lax.axis_index("c")
    lanes = lax.broadcasted_iota(jnp.int32, (16,), 0)
    zero16 = jnp.zeros((16,), jnp.float32)

    def _zz(i, _):
        zslab[i >> 3, pl.ds((i & 7) * 16, 16)] = zero16
        return 0

    def _zv(i, _):
        vslab[i >> 3, pl.ds((i & 7) * 16, 16)] = zero16
        return 0

    lax.fori_loop(0, _TROWS * 8, _zz, 0)
    for bi in range(2):
        b = wid * 2 + bi
        pltpu.sync_copy(idx_hbm.at[b, 0], ibuf)
        idx = ibuf[...][0]
        grp = idx // 128
        q = idx - grp * 128
        sg = q // 16
        lane = q - sg * 16
        mask = jnp.where(lanes == lane, jnp.full((16,), 1.0, jnp.float32), zero16)
        lax.fori_loop(0, _TROWS * 8, _zv, 0)
        off_w = pl.multiple_of(grp * 128, 128)
        off_z = pl.multiple_of((1 - grp) * 128, 128)
        for c in range(_NCHUNK):
            rows = pl.ds(c * _TROWS, _TROWS)
            pltpu.sync_copy(x_hbm.at[b, rows, pl.ds(off_w, 128)], xbuf)
            for j in range(8):

                @pl.when(sg == j)
                def _(j=j):
                    def _ins(r, _):
                        vslab[r, pl.ds(j * 16, 16)] = (
                            xbuf[r, pl.ds(j * 16, 16)] * mask)
                        return 0

                    lax.fori_loop(0, _TROWS, _ins, 0)

            pltpu.sync_copy(vslab, y_hbm.at[b, rows, pl.ds(off_w, 128)])
            pltpu.sync_copy(zslab, y_hbm.at[b, rows, pl.ds(off_z, 128)])


_sc_y = pl.kernel(
    _sc_y_body,
    out_type=jax.ShapeDtypeStruct((_B, _L, _K), jnp.float32),
    mesh=plsc.VectorSubcoreMesh(core_axis_name="c", subcore_axis_name="s"),
    scratch_types=[
        pltpu.VMEM((16,), jnp.int32),
        pltpu.VMEM((_TROWS, 128), jnp.float32),
        pltpu.VMEM((_TROWS, 128), jnp.float32),
        pltpu.VMEM((_TROWS, 128), jnp.float32),
    ],
)


@jax.jit
def kernel(spikes):
    ff3, idx3, w3 = pl.pallas_call(
        _prefix_body,
        grid=(1,),
        in_specs=[pl.BlockSpec((_B, _PRE, _K), lambda i: (0, 0, 0))],
        out_specs=[
            pl.BlockSpec((_B, 1, 1), lambda i: (0, 0, 0)),
            pl.BlockSpec((_B, 1, 1), lambda i: (0, 0, 0)),
            pl.BlockSpec((_B, 1, _K), lambda i: (0, 0, 0)),
        ],
        out_shape=[
            jax.ShapeDtypeStruct((_B, 1, 1), jnp.int32),
            jax.ShapeDtypeStruct((_B, 1, 1), jnp.int32),
            jax.ShapeDtypeStruct((_B, 1, _K), jnp.float32),
        ],
    )(spikes)
    ff = ff3[:, 0, 0]
    allfound = jnp.all(ff < _BIG)
    idxv = lax.rem(ff, _K)

    def _sc_path(x, i):
        im = jnp.broadcast_to(i.reshape(_B, 1, 1), (_B, 1, 16))
        return idx3[:, 0, 0], w3[:, 0, :], _sc_y(x, im)

    return lax.cond(allfound, _sc_path, lambda x, i: _full_path(x), spikes, idxv)


# speedup vs baseline: 1.0267x; 1.0267x over previous
import functools

import jax
import jax.numpy as jnp
from jax import lax
from jax.experimental import pallas as pl
from jax.experimental.pallas import tpu as pltpu
from jax.experimental.pallas import tpu_sc as plsc

_B, _L, _K = 64, 2048, 256
_THR = 0.5
_BIG = 1 << 30


_PRE = 8


def _wta_body(x_ref, idx_ref, w_ref, y_ref, idx_s):
    x = x_ref[0]
    kk1 = lax.broadcasted_iota(jnp.int32, (1, _K), 1)
    xp = x[0:_PRE, :]
    iip = lax.broadcasted_iota(jnp.int32, (_PRE, _K), 0)
    kkp = lax.broadcasted_iota(jnp.int32, (_PRE, _K), 1)
    ffp = jnp.min(jnp.where(xp > _THR, iip * _K + kkp, _BIG))

    @pl.when(ffp < _BIG)
    def _():
        idx_s[0] = lax.rem(ffp, _K)

    @pl.when(ffp >= _BIG)
    def _():
        s = x > _THR
        ii = lax.broadcasted_iota(jnp.int32, (_L, _K), 0)
        kk = lax.broadcasted_iota(jnp.int32, (_L, _K), 1)
        ff = jnp.min(jnp.where(s, ii * _K + kk, _BIG))
        total = jnp.sum(x, axis=0, keepdims=True)
        maxv = jnp.max(total)
        k_fb = jnp.min(jnp.where(total == maxv, kk1, _BIG))
        idx_s[0] = jnp.where(ff < _BIG, lax.rem(ff, _K), k_fb)

    idx = idx_s[0]
    w = (kk1 == idx).astype(jnp.float32)
    idx_ref[0] = jnp.full((1, 1), idx, jnp.int32)
    w_ref[0] = w
    y_ref[0] = x * w


def _full_path(spikes):
    idx3, w3, y = pl.pallas_call(
        _wta_body,
        grid=(_B,),
        in_specs=[pl.BlockSpec((1, _L, _K), lambda b: (b, 0, 0))],
        out_specs=[
            pl.BlockSpec((1, 1, 1), lambda b: (b, 0, 0)),
            pl.BlockSpec((1, 1, _K), lambda b: (b, 0, 0)),
            pl.BlockSpec((1, _L, _K), lambda b: (b, 0, 0)),
        ],
        out_shape=[
            jax.ShapeDtypeStruct((_B, 1, 1), jnp.int32),
            jax.ShapeDtypeStruct((_B, 1, _K), jnp.float32),
            jax.ShapeDtypeStruct((_B, _L, _K), jnp.float32),
        ],
        scratch_shapes=[pltpu.SMEM((1,), jnp.int32)],
    )(spikes)
    return idx3[:, 0, 0], w3[:, 0, :], y


def _prefix_body(x_ref, ff_ref, idx_ref, w_ref):
    x = x_ref[...]
    ii = lax.broadcasted_iota(jnp.int32, (_B, _PRE, _K), 1)
    kk = lax.broadcasted_iota(jnp.int32, (_B, _PRE, _K), 2)
    ff = jnp.min(jnp.where(x > _THR, ii * _K + kk, _BIG), axis=(1, 2))
    ff_ref[...] = ff.reshape(_B, 1, 1)
    idx = lax.rem(ff, _K).reshape(_B, 1, 1)
    idx_ref[...] = idx
    kk3 = lax.broadcasted_iota(jnp.int32, (_B, 1, _K), 2)
    w_ref[...] = (kk3 == idx).astype(jnp.float32)


_GRPS = _K // 128


def _mask_body(idxp_ref, x_ref, idx_ref, w_ref, y_ref):
    b = pl.program_id(0)
    idx = idxp_ref[b]
    grp = idx // 128
    base = pl.multiple_of(grp * 128, 128)
    obase = pl.multiple_of((1 - grp) * 128, 128)
    lanei = lax.broadcasted_iota(jnp.int32, (1, 128), 1)
    wrow = (lanei == idx - base).astype(jnp.float32)
    y_ref[0, :, pl.ds(obase, 128)] = jnp.zeros((_L, 128), jnp.float32)
    y_ref[0, :, pl.ds(base, 128)] = x_ref[0] * wrow
    w_ref[0, :, pl.ds(obase, 128)] = jnp.zeros((1, 128), jnp.float32)
    w_ref[0, :, pl.ds(base, 128)] = wrow
    idx_ref[0] = jnp.full((1, 1), idx, jnp.int32)


def _cheap_path(spikes, idxv):
    grid_spec = pltpu.PrefetchScalarGridSpec(
        num_scalar_prefetch=1,
        grid=(_B,),
        in_specs=[
            pl.BlockSpec((1, _L, 128), lambda b, idxp: (b, 0, idxp[b] // 128)),
        ],
        out_specs=[
            pl.BlockSpec((1, 1, 1), lambda b, idxp: (b, 0, 0)),
            pl.BlockSpec((1, 1, _K), lambda b, idxp: (b, 0, 0)),
            pl.BlockSpec((1, _L, _K), lambda b, idxp: (b, 0, 0)),
        ],
    )
    idx3, w3, y = pl.pallas_call(
        _mask_body,
        grid_spec=grid_spec,
        out_shape=[
            jax.ShapeDtypeStruct((_B, 1, 1), jnp.int32),
            jax.ShapeDtypeStruct((_B, 1, _K), jnp.float32),
            jax.ShapeDtypeStruct((_B, _L, _K), jnp.float32),
        ],
    )(idxv, spikes)
    return idx3[:, 0, 0], w3[:, 0, :], y


_NW = 32
_TROWS = 256
_NCHUNK = _L // _TROWS


def _sc_y_body(x_hbm, idx_hbm, y_hbm, ibuf, xbuf, vslab, zslab):
    wid = lax.axis_index("s") * 2 + lax.axis_index("c")
    lanes = lax.broadcasted_iota(jnp.int32, (16,), 0)
    zero16 = jnp.zeros((16,), jnp.float32)

    def _zz(i, _):
        zslab[i >> 3, pl.ds((i & 7) * 16, 16)] = zero16
        return 0

    def _zv(i, _):
        vslab[i >> 3, pl.ds((i & 7) * 16, 16)] = zero16
        return 0

    lax.fori_loop(0, _TROWS * 8, _zz, 0)
    for bi in range(2):
        b = wid * 2 + bi
        pltpu.sync_copy(idx_hbm.at[b, 0], ibuf)
        idx = ibuf[...][0]
        grp = idx // 128
        q = idx - grp * 128
        sg = q // 16
        lane = q - sg * 16
        mask = jnp.where(lanes == lane, jnp.full((16,), 1.0, jnp.float32), zero16)
        lax.fori_loop(0, _TROWS * 8, _zv, 0)
        off_w = pl.multiple_of(grp * 128, 128)
        off_z = pl.multiple_of((1 - grp) * 128, 128)
        for c in range(_NCHUNK):
            rows = pl.ds(c * _TROWS, _TROWS)
            pltpu.sync_copy(x_hbm.at[b, rows, pl.ds(off_w, 128)], xbuf)
            for j in range(8):

                @pl.when(sg == j)
                def _(j=j):
                    def _ins(r, _):
                        vslab[r, pl.ds(j * 16, 16)] = (
                            xbuf[r, pl.ds(j * 16, 16)] * mask)
                        return 0

                    lax.fori_loop(0, _TROWS, _ins, 0)

            pltpu.sync_copy(vslab, y_hbm.at[b, rows, pl.ds(off_w, 128)])
            pltpu.sync_copy(zslab, y_hbm.at[b, rows, pl.ds(off_z, 128)])


_sc_y = pl.kernel(
    _sc_y_body,
    out_type=jax.ShapeDtypeStruct((_B, _L, _K), jnp.float32),
    mesh=plsc.VectorSubcoreMesh(core_axis_name="c", subcore_axis_name="s"),
    scratch_types=[
        pltpu.VMEM((16,), jnp.int32),
        pltpu.VMEM((_TROWS, 128), jnp.float32),
        pltpu.VMEM((_TROWS, 128), jnp.float32),
        pltpu.VMEM((_TROWS, 128), jnp.float32),
    ],
)


@jax.jit
def kernel(spikes):
    ff3, idx3, w3 = pl.pallas_call(
        _prefix_body,
        grid=(1,),
        in_specs=[pl.BlockSpec((_B, _PRE, _K), lambda i: (0, 0, 0))],
        out_specs=[
            pl.BlockSpec((_B, 1, 1), lambda i: (0, 0, 0)),
            pl.BlockSpec((_B, 1, 1), lambda i: (0, 0, 0)),
            pl.BlockSpec((_B, 1, _K), lambda i: (0, 0, 0)),
        ],
        out_shape=[
            jax.ShapeDtypeStruct((_B, 1, 1), jnp.int32),
            jax.ShapeDtypeStruct((_B, 1, 1), jnp.int32),
            jax.ShapeDtypeStruct((_B, 1, _K), jnp.float32),
        ],
    )(spikes)
    ff = ff3[:, 0, 0]
    allfound = jnp.all(ff < _BIG)
    idxv = lax.rem(ff, _K)

    def _sc_path(x, i):
        im = jnp.broadcast_to(i.reshape(_B, 1, 1), (_B, 1, 16))
        return idx3[:, 0, 0], w3[:, 0, :], _sc_y(x, im)

    return lax.cond(allfound, _sc_path, lambda x, i: _full_path(x), spikes, idxv)


# speedup vs baseline: 1.2263x; 1.1944x over previous
import functools

import jax
import jax.numpy as jnp
from jax import lax
from jax.experimental import pallas as pl
from jax.experimental.pallas import tpu as pltpu
from jax.experimental.pallas import tpu_sc as plsc

_B, _L, _K = 64, 2048, 256
_THR = 0.5
_BIG = 1 << 30


_PRE = 8


def _wta_body(x_ref, idx_ref, w_ref, y_ref, idx_s):
    x = x_ref[0]
    kk1 = lax.broadcasted_iota(jnp.int32, (1, _K), 1)
    xp = x[0:_PRE, :]
    iip = lax.broadcasted_iota(jnp.int32, (_PRE, _K), 0)
    kkp = lax.broadcasted_iota(jnp.int32, (_PRE, _K), 1)
    ffp = jnp.min(jnp.where(xp > _THR, iip * _K + kkp, _BIG))

    @pl.when(ffp < _BIG)
    def _():
        idx_s[0] = lax.rem(ffp, _K)

    @pl.when(ffp >= _BIG)
    def _():
        s = x > _THR
        ii = lax.broadcasted_iota(jnp.int32, (_L, _K), 0)
        kk = lax.broadcasted_iota(jnp.int32, (_L, _K), 1)
        ff = jnp.min(jnp.where(s, ii * _K + kk, _BIG))
        total = jnp.sum(x, axis=0, keepdims=True)
        maxv = jnp.max(total)
        k_fb = jnp.min(jnp.where(total == maxv, kk1, _BIG))
        idx_s[0] = jnp.where(ff < _BIG, lax.rem(ff, _K), k_fb)

    idx = idx_s[0]
    w = (kk1 == idx).astype(jnp.float32)
    idx_ref[0] = jnp.full((1, 1), idx, jnp.int32)
    w_ref[0] = w
    y_ref[0] = x * w


def _full_path(spikes):
    idx3, w3, y = pl.pallas_call(
        _wta_body,
        grid=(_B,),
        in_specs=[pl.BlockSpec((1, _L, _K), lambda b: (b, 0, 0))],
        out_specs=[
            pl.BlockSpec((1, 1, 1), lambda b: (b, 0, 0)),
            pl.BlockSpec((1, 1, _K), lambda b: (b, 0, 0)),
            pl.BlockSpec((1, _L, _K), lambda b: (b, 0, 0)),
        ],
        out_shape=[
            jax.ShapeDtypeStruct((_B, 1, 1), jnp.int32),
            jax.ShapeDtypeStruct((_B, 1, _K), jnp.float32),
            jax.ShapeDtypeStruct((_B, _L, _K), jnp.float32),
        ],
        scratch_shapes=[pltpu.SMEM((1,), jnp.int32)],
    )(spikes)
    return idx3[:, 0, 0], w3[:, 0, :], y


def _prefix_body(x_ref, ff_ref, idx_ref, w_ref):
    x = x_ref[...]
    ii = lax.broadcasted_iota(jnp.int32, (_B, _PRE, _K), 1)
    kk = lax.broadcasted_iota(jnp.int32, (_B, _PRE, _K), 2)
    ff = jnp.min(jnp.where(x > _THR, ii * _K + kk, _BIG), axis=(1, 2))
    ff_ref[...] = ff.reshape(_B, 1, 1)
    idx = lax.rem(ff, _K).reshape(_B, 1, 1)
    idx_ref[...] = idx
    kk3 = lax.broadcasted_iota(jnp.int32, (_B, 1, _K), 2)
    w_ref[...] = (kk3 == idx).astype(jnp.float32)


_GRPS = _K // 128


def _mask_body(idxp_ref, x_ref, idx_ref, w_ref, y_ref):
    b = pl.program_id(0)
    idx = idxp_ref[b]
    grp = idx // 128
    base = pl.multiple_of(grp * 128, 128)
    obase = pl.multiple_of((1 - grp) * 128, 128)
    lanei = lax.broadcasted_iota(jnp.int32, (1, 128), 1)
    wrow = (lanei == idx - base).astype(jnp.float32)
    y_ref[0, :, pl.ds(obase, 128)] = jnp.zeros((_L, 128), jnp.float32)
    y_ref[0, :, pl.ds(base, 128)] = x_ref[0] * wrow
    w_ref[0, :, pl.ds(obase, 128)] = jnp.zeros((1, 128), jnp.float32)
    w_ref[0, :, pl.ds(base, 128)] = wrow
    idx_ref[0] = jnp.full((1, 1), idx, jnp.int32)


def _cheap_path(spikes, idxv):
    grid_spec = pltpu.PrefetchScalarGridSpec(
        num_scalar_prefetch=1,
        grid=(_B,),
        in_specs=[
            pl.BlockSpec((1, _L, 128), lambda b, idxp: (b, 0, idxp[b] // 128)),
        ],
        out_specs=[
            pl.BlockSpec((1, 1, 1), lambda b, idxp: (b, 0, 0)),
            pl.BlockSpec((1, 1, _K), lambda b, idxp: (b, 0, 0)),
            pl.BlockSpec((1, _L, _K), lambda b, idxp: (b, 0, 0)),
        ],
    )
    idx3, w3, y = pl.pallas_call(
        _mask_body,
        grid_spec=grid_spec,
        out_shape=[
            jax.ShapeDtypeStruct((_B, 1, 1), jnp.int32),
            jax.ShapeDtypeStruct((_B, 1, _K), jnp.float32),
            jax.ShapeDtypeStruct((_B, _L, _K), jnp.float32),
        ],
    )(idxv, spikes)
    return idx3[:, 0, 0], w3[:, 0, :], y


_NW = 32
_TROWS = 128
_NCHUNK = _L // _TROWS


def _sc_y_body(x_hbm, idx_hbm, y_hbm, ibuf, xbuf0, xbuf1, vslab0, vslab1,
               zslab, sem_in0, sem_in1, sem_v0, sem_v1, sem_z0, sem_z1):
    wid = lax.axis_index("s") * 2 + lax.axis_index("c")
    lanes = lax.broadcasted_iota(jnp.int32, (16,), 0)
    zero16 = jnp.zeros((16,), jnp.float32)
    xbufs = (xbuf0, xbuf1)
    vslabs = (vslab0, vslab1)
    sems_in = (sem_in0, sem_in1)
    sems_v = (sem_v0, sem_v1)
    sems_z = (sem_z0, sem_z1)

    def _mkzero(slab):
        def _z(i, _):
            slab[i >> 3, pl.ds((i & 7) * 16, 16)] = zero16
            return 0
        return _z

    lax.fori_loop(0, _TROWS * 8, _mkzero(zslab), 0)
    out_handles = [None, None]
    for bi in range(2):
        b = wid * 2 + bi
        pltpu.sync_copy(idx_hbm.at[b, 0], ibuf)
        idx = ibuf[...][0]
        grp = idx // 128
        q = idx - grp * 128
        sg = q // 16
        lane = q - sg * 16
        mask = jnp.where(lanes == lane, jnp.full((16,), 1.0, jnp.float32), zero16)
        off_w = pl.multiple_of(grp * 128, 128)
        off_z = pl.multiple_of((1 - grp) * 128, 128)
        in_handles = [None, None]
        in_handles[0] = pltpu.async_copy(
            x_hbm.at[b, pl.ds(0, _TROWS), pl.ds(off_w, 128)], xbufs[0],
            sems_in[0])
        for c in range(_NCHUNK):
            p = c % 2
            rows = pl.ds(c * _TROWS, _TROWS)
            in_handles[p].wait()
            if c + 1 < _NCHUNK:
                nrows = pl.ds((c + 1) * _TROWS, _TROWS)
                in_handles[1 - p] = pltpu.async_copy(
                    x_hbm.at[b, nrows, pl.ds(off_w, 128)], xbufs[1 - p],
                    sems_in[1 - p])
            if out_handles[p] is not None:
                hv, hz = out_handles[p]
                hv.wait()
                hz.wait()
            if c < 2:
                lax.fori_loop(0, _TROWS * 8, _mkzero(vslabs[p]), 0)
            for j in range(8):

                @pl.when(sg == j)
                def _(j=j, p=p):
                    def _ins(r, _):
                        vslabs[p][r, pl.ds(j * 16, 16)] = (
                            xbufs[p][r, pl.ds(j * 16, 16)] * mask)
                        return 0

                    lax.fori_loop(0, _TROWS, _ins, 0)

            hv = pltpu.async_copy(
                vslabs[p], y_hbm.at[b, rows, pl.ds(off_w, 128)], sems_v[p])
            hz = pltpu.async_copy(
                zslab, y_hbm.at[b, rows, pl.ds(off_z, 128)], sems_z[p])
            out_handles[p] = (hv, hz)
    for p in range(2):
        if out_handles[p] is not None:
            hv, hz = out_handles[p]
            hv.wait()
            hz.wait()


_sc_y = pl.kernel(
    _sc_y_body,
    out_type=jax.ShapeDtypeStruct((_B, _L, _K), jnp.float32),
    mesh=plsc.VectorSubcoreMesh(core_axis_name="c", subcore_axis_name="s"),
    scratch_types=[
        pltpu.VMEM((16,), jnp.int32),
        pltpu.VMEM((_TROWS, 128), jnp.float32),
        pltpu.VMEM((_TROWS, 128), jnp.float32),
        pltpu.VMEM((_TROWS, 128), jnp.float32),
        pltpu.VMEM((_TROWS, 128), jnp.float32),
        pltpu.VMEM((_TROWS, 128), jnp.float32),
        pltpu.SemaphoreType.DMA,
        pltpu.SemaphoreType.DMA,
        pltpu.SemaphoreType.DMA,
        pltpu.SemaphoreType.DMA,
        pltpu.SemaphoreType.DMA,
        pltpu.SemaphoreType.DMA,
    ],
)


@jax.jit
def kernel(spikes):
    ff3, idx3, w3 = pl.pallas_call(
        _prefix_body,
        grid=(1,),
        in_specs=[pl.BlockSpec((_B, _PRE, _K), lambda i: (0, 0, 0))],
        out_specs=[
            pl.BlockSpec((_B, 1, 1), lambda i: (0, 0, 0)),
            pl.BlockSpec((_B, 1, 1), lambda i: (0, 0, 0)),
            pl.BlockSpec((_B, 1, _K), lambda i: (0, 0, 0)),
        ],
        out_shape=[
            jax.ShapeDtypeStruct((_B, 1, 1), jnp.int32),
            jax.ShapeDtypeStruct((_B, 1, 1), jnp.int32),
            jax.ShapeDtypeStruct((_B, 1, _K), jnp.float32),
        ],
    )(spikes)
    ff = ff3[:, 0, 0]
    allfound = jnp.all(ff < _BIG)
    idxv = lax.rem(ff, _K)

    def _sc_path(x, i):
        im = jnp.broadcast_to(i.reshape(_B, 1, 1), (_B, 1, 16))
        return idx3[:, 0, 0], w3[:, 0, :], _sc_y(x, im)

    return lax.cond(allfound, _sc_path, lambda x, i: _full_path(x), spikes, idxv)
